# baseline (speedup 1.0000x reference)
import jax
import jax.numpy as jnp
from jax import lax
from jax.experimental import pallas as pl
from jax.experimental.pallas import tpu as pltpu
from jax.experimental.pallas import tpu_sc as plsc

B = 1024
NV = 2000
NC = 8000
K = 3

LANES = 16
NUM_CORES = 2
NUM_SUBCORES = 16
NW = NUM_CORES * NUM_SUBCORES
BPW = B // NW
NG = NC // LANES
CPR = NV // LANES
HB = BPW // 2


def _sc_body(inp_hbm, emb_hbm, ci_hbm, cs_hbm, out_hbm,
             idxv, tbl, civ, csv, accs, outv, sem, sem2):
    w = lax.axis_index("s") * NUM_CORES + lax.axis_index("c")
    base = w * BPW

    pltpu.sync_copy(inp_hbm.at[pl.ds(base, BPW)], idxv)
    tbl_cp = pltpu.async_copy(emb_hbm.at[idxv], tbl, sem)
    ci_cp = pltpu.async_copy(ci_hbm, civ, sem2)
    cs_cp = pltpu.async_copy(cs_hbm, csv, sem2)
    ci_cp.wait()
    cs_cp.wait()
    tbl_cp.wait()

    def _sig(c, carry):
        sl = pl.ds(c * LANES, LANES)
        for b in range(BPW):
            z = tbl[b, sl]
            tbl[b, sl] = 1.0 / (1.0 + jnp.exp(-z))
        return carry
    lax.fori_loop(0, CPR, _sig, None)

    one = jnp.full((LANES,), 1.0, jnp.float32)
    lane3 = lax.iota(jnp.int32, LANES) * K
    for half in range(2):
        def _grp(g, acc_c, half=half):
            gbase = lane3 + g * (LANES * K)
            id0 = plsc.load_gather(civ, [gbase])
            id1 = plsc.load_gather(civ, [gbase + 1])
            id2 = plsc.load_gather(civ, [gbase + 2])
            sg0 = plsc.load_gather(csv, [gbase])
            sg1 = plsc.load_gather(csv, [gbase + 1])
            sg2 = plsc.load_gather(csv, [gbase + 2])
            a0 = 0.5 + 0.5 * sg0
            a1 = 0.5 + 0.5 * sg1
            a2 = 0.5 + 0.5 * sg2
            nxt = []
            for j in range(HB):
                row = tbl.at[half * HB + j]
                l0 = plsc.load_gather(row, [id0])
                l1 = plsc.load_gather(row, [id1])
                l2 = plsc.load_gather(row, [id2])
                t = (a0 - sg0 * l0) * (a1 - sg1 * l1) * (a2 - sg2 * l2)
                nxt.append(acc_c[j] * (1.0 - t))
            return tuple(nxt)
        fin = lax.fori_loop(0, NG, _grp, (one,) * HB)
        for j in range(HB):
            accs[pl.ds((half * HB + j) * LANES, LANES)] = fin[j]

    lane = lax.iota(jnp.int32, LANES)
    for half in range(2):
        bidx = lane * LANES + half * (LANES * LANES)
        p = plsc.load_gather(accs, [bidx])
        for l in range(1, LANES):
            p = p * plsc.load_gather(accs, [bidx + l])
        outv[pl.ds(half * LANES, LANES)] = p

    pltpu.sync_copy(outv, out_hbm.at[pl.ds(base, BPW)])


def kernel(input, emb_weight, clause_idx, clause_sign):
    inp = input.astype(jnp.int32)
    ci = clause_idx.astype(jnp.int32).reshape(NC * K)
    cs = clause_sign.astype(jnp.float32).reshape(NC * K)

    mesh = plsc.VectorSubcoreMesh(
        core_axis_name="c", subcore_axis_name="s",
        num_cores=NUM_CORES, num_subcores=NUM_SUBCORES)
    f = pl.kernel(
        _sc_body,
        out_type=jax.ShapeDtypeStruct((B,), jnp.float32),
        mesh=mesh,
        compiler_params=pltpu.CompilerParams(
            use_tc_tiling_on_sc=False, needs_layout_passes=False),
        scratch_types=[
            pltpu.VMEM((BPW,), jnp.int32),
            pltpu.VMEM((BPW, NV), jnp.float32),
            pltpu.VMEM((NC * K,), jnp.int32),
            pltpu.VMEM((NC * K,), jnp.float32),
            pltpu.VMEM((BPW * LANES,), jnp.float32),
            pltpu.VMEM((BPW,), jnp.float32),
            pltpu.SemaphoreType.DMA,
            pltpu.SemaphoreType.DMA,
        ],
    )
    return f(inp, emb_weight.astype(jnp.float32), ci, cs)
